# Initial kernel scaffold; baseline (speedup 1.0000x reference)
import jax
import jax.numpy as jnp
from jax.experimental import pallas as pl


def kernel(x, edge_index, W0, b0, W1, b1, W2, b2, W3, b3, Wl, bl):
    raise NotImplementedError("write your pallas kernel here")


# speedup vs baseline: 5.6945x; 5.6945x over previous
import functools

import jax
import jax.numpy as jnp
from jax import lax
from jax.experimental import pallas as pl
from jax.experimental.pallas import tpu as pltpu
from jax.experimental.pallas import tpu_sc as plsc

N = 10000
E = 320000
D = 128
H = 128
C = 40

NC = 2
NS = 16
NW = NC * NS
CHUNK = 128
NCHUNK = -(-E // (NW * CHUNK))
EPT = NCHUNK * CHUNK
EPAD = EPT * NW
NP = 10240
RPT = NP // NS
BN = 1024


def _mesh():
    return plsc.VectorSubcoreMesh(
        core_axis_name="c", subcore_axis_name="s", num_cores=NC, num_subcores=NS
    )


@functools.cache
def _get_deg_kernel():
    @functools.partial(
        pl.kernel,
        out_type=[
            jax.ShapeDtypeStruct((NC, NP, H), jnp.float32),
            jax.ShapeDtypeStruct((NC, NP, H), jnp.float32),
        ],
        mesh=_mesh(),
        scratch_types=[
            pltpu.VMEM((NCHUNK, CHUNK), jnp.int32),
            pltpu.VMEM((NCHUNK, CHUNK), jnp.int32),
            pltpu.VMEM((CHUNK, H), jnp.float32),
            pltpu.VMEM_SHARED((NP, H), jnp.float32),
        ],
    )
    def _deg_kernel(src_hbm, dst_hbm, ones_hbm, zf_hbm,
                    dego_hbm, degi_hbm, sidx, didx, ones_v, acc):
        c = lax.axis_index("c")
        s = lax.axis_index("s")
        wid = s * NC + c
        pltpu.sync_copy(zf_hbm, acc.at[pl.ds(s * RPT, RPT)])
        pltpu.sync_copy(ones_hbm, ones_v)
        pltpu.sync_copy(src_hbm.at[wid], sidx)
        pltpu.sync_copy(dst_hbm.at[wid], didx)
        plsc.subcore_barrier()

        def body_s(j, carry):
            pltpu.sync_copy(ones_v, acc.at[sidx.at[j]], add=True)
            return carry

        lax.fori_loop(0, NCHUNK, body_s, 0)
        plsc.subcore_barrier()
        pltpu.sync_copy(acc.at[pl.ds(s * RPT, RPT)],
                        dego_hbm.at[c, pl.ds(s * RPT, RPT)])
        pltpu.sync_copy(zf_hbm, acc.at[pl.ds(s * RPT, RPT)])
        plsc.subcore_barrier()

        def body_d(j, carry):
            pltpu.sync_copy(ones_v, acc.at[didx.at[j]], add=True)
            return carry

        lax.fori_loop(0, NCHUNK, body_d, 0)
        plsc.subcore_barrier()
        pltpu.sync_copy(acc.at[pl.ds(s * RPT, RPT)],
                        degi_hbm.at[c, pl.ds(s * RPT, RPT)])

    return _deg_kernel


@functools.cache
def _get_agg_kernel():
    @functools.partial(
        pl.kernel,
        out_type=jax.ShapeDtypeStruct((NC, NP, H), jnp.float32),
        mesh=_mesh(),
        scratch_types=[
            pltpu.VMEM((NCHUNK, CHUNK), jnp.int32),
            pltpu.VMEM((NCHUNK, CHUNK), jnp.int32),
            pltpu.VMEM((CHUNK, H), jnp.float32),
            pltpu.VMEM_SHARED((NP, H), jnp.float32),
            pltpu.SemaphoreType.DMA,
        ],
    )
    def _agg_kernel(hw_hbm, src_hbm, dst_hbm, zf_hbm,
                    out_hbm, sidx, didx, rows, acc, sem):
        c = lax.axis_index("c")
        s = lax.axis_index("s")
        wid = s * NC + c
        pltpu.sync_copy(zf_hbm, acc.at[pl.ds(s * RPT, RPT)])
        pltpu.sync_copy(src_hbm.at[wid], sidx)
        pltpu.sync_copy(dst_hbm.at[wid], didx)
        plsc.subcore_barrier()

        def body(j, carry):
            pltpu.async_copy(hw_hbm.at[sidx.at[j]], rows, sem).wait()
            pltpu.sync_copy(rows, acc.at[didx.at[j]], add=True)
            return carry

        lax.fori_loop(0, NCHUNK, body, 0)
        plsc.subcore_barrier()
        pltpu.sync_copy(acc.at[pl.ds(s * RPT, RPT)],
                        out_hbm.at[c, pl.ds(s * RPT, RPT)])

    return _agg_kernel


def _norm_col(degp):
    return lax.rsqrt(jnp.maximum((degp[0] + degp[1])[:, :1], 1.0))


def _tc_pre_body(x_ref, w_ref, dego_ref, hw_ref):
    ns = _norm_col(dego_ref[...])
    hw_ref[...] = (
        jnp.dot(x_ref[...], w_ref[...], preferred_element_type=jnp.float32) * ns
    )


def _tc_mid_body(accp_ref, degi_ref, b_ref, w_ref, dego_ref, h_ref, hw_ref):
    acc = accp_ref[0] + accp_ref[1]
    nd = _norm_col(degi_ref[...])
    h = jnp.maximum(acc * nd + b_ref[...], 0.0)
    h_ref[...] = h
    ns = _norm_col(dego_ref[...])
    hw_ref[...] = (
        jnp.dot(h, w_ref[...], preferred_element_type=jnp.float32) * ns
    )


def _tc_fin_body(accp_ref, degi_ref, b_ref, h0_ref, h1_ref, h2_ref,
                 wl_ref, bl_ref, out_ref):
    acc = accp_ref[0] + accp_ref[1]
    nd = _norm_col(degi_ref[...])
    h3 = jnp.maximum(acc * nd + b_ref[...], 0.0)
    wl = wl_ref[...]
    out = jnp.dot(h0_ref[...], wl[0:H], preferred_element_type=jnp.float32)
    out += jnp.dot(h1_ref[...], wl[H:2 * H], preferred_element_type=jnp.float32)
    out += jnp.dot(h2_ref[...], wl[2 * H:3 * H], preferred_element_type=jnp.float32)
    out += jnp.dot(h3, wl[3 * H:4 * H], preferred_element_type=jnp.float32)
    out_ref[...] = out + bl_ref[...]


_GRID = NP // BN

_feat_spec = pl.BlockSpec((BN, H), lambda i: (i, 0))
_pair_spec = pl.BlockSpec((2, BN, H), lambda i: (0, i, 0))
_deg_spec = pl.BlockSpec((2, BN, H), lambda i: (0, i, 0))
_w_spec = pl.BlockSpec((H, H), lambda i: (0, 0))
_b_spec = pl.BlockSpec((1, H), lambda i: (0, 0))

_tc_pre = pl.pallas_call(
    _tc_pre_body,
    grid=(_GRID,),
    in_specs=[_feat_spec, _w_spec, _deg_spec],
    out_specs=_feat_spec,
    out_shape=jax.ShapeDtypeStruct((NP, H), jnp.float32),
)

_tc_mid = pl.pallas_call(
    _tc_mid_body,
    grid=(_GRID,),
    in_specs=[_pair_spec, _deg_spec, _b_spec, _w_spec, _deg_spec],
    out_specs=[_feat_spec, _feat_spec],
    out_shape=[
        jax.ShapeDtypeStruct((NP, H), jnp.float32),
        jax.ShapeDtypeStruct((NP, H), jnp.float32),
    ],
)

_tc_fin = pl.pallas_call(
    _tc_fin_body,
    grid=(_GRID,),
    in_specs=[_pair_spec, _deg_spec, _b_spec, _feat_spec, _feat_spec,
              _feat_spec, pl.BlockSpec((4 * H, H), lambda i: (0, 0)), _b_spec],
    out_specs=_feat_spec,
    out_shape=jax.ShapeDtypeStruct((NP, H), jnp.float32),
)


def kernel(x, edge_index, W0, b0, W1, b1, W2, b2, W3, b3, Wl, bl):
    f32 = jnp.float32
    xp = jnp.zeros((NP, D), f32).at[:N].set(x)
    pad = jnp.full((EPAD - E,), N, jnp.int32)
    srcp = jnp.concatenate([edge_index[0], pad]).reshape(NW, NCHUNK, CHUNK)
    dstp = jnp.concatenate([edge_index[1], pad]).reshape(NW, NCHUNK, CHUNK)
    zf = jnp.zeros((RPT, H), f32)
    ones128 = jnp.ones((CHUNK, H), f32)
    wlp = jnp.zeros((4 * H, H), f32).at[:, :C].set(Wl)
    blp = jnp.zeros((1, H), f32).at[0, :C].set(bl)

    _deg_kernel = _get_deg_kernel()
    _agg_kernel = _get_agg_kernel()
    dego, degi = _deg_kernel(srcp, dstp, ones128, zf)

    hs = []
    hw = _tc_pre(xp, W0, dego)
    for (bprev, W) in ((b0, W1), (b1, W2), (b2, W3)):
        accp = _agg_kernel(hw, srcp, dstp, zf)
        h, hw = _tc_mid(accp, degi, bprev.reshape(1, H), W, dego)
        hs.append(h)
    accp = _agg_kernel(hw, srcp, dstp, zf)
    outp = _tc_fin(accp, degi, b3.reshape(1, H), hs[0], hs[1], hs[2], wlp, blp)
    return outp[:N, :C]


# speedup vs baseline: 9.5045x; 1.6691x over previous
import functools

import jax
import jax.numpy as jnp
from jax import lax
from jax.experimental import pallas as pl
from jax.experimental.pallas import tpu as pltpu
from jax.experimental.pallas import tpu_sc as plsc

N = 10000
E = 320000
D = 128
H = 128
C = 40

NC = 2
NS = 16
NW = NC * NS
CHUNK = 128
NBUF = 1
NCHUNK = 80
EPT = NCHUNK * CHUNK
EPAD = EPT * NW
NGCHUNK = NCHUNK + NBUF
NGROUP = NCHUNK // NBUF
NP = 10240
NPS = 10112
RPT = NPS // NS
BN = 1024


def _mesh():
    return plsc.VectorSubcoreMesh(
        core_axis_name="c", subcore_axis_name="s", num_cores=NC, num_subcores=NS
    )


@functools.cache
def _get_deg_kernel():
    @functools.partial(
        pl.kernel,
        out_type=[
            jax.ShapeDtypeStruct((NC, NP, H), jnp.float32),
            jax.ShapeDtypeStruct((NC, NP, H), jnp.float32),
        ],
        mesh=_mesh(),
        scratch_types=[
            pltpu.VMEM((NGCHUNK, CHUNK), jnp.int32),
            pltpu.VMEM((NCHUNK, CHUNK), jnp.int32),
            pltpu.VMEM((CHUNK, H), jnp.float32),
            pltpu.VMEM_SHARED((NPS, H), jnp.float32),
            pltpu.SemaphoreType.DMA((NBUF,)),
        ],
    )
    def _deg_kernel(src_hbm, dst_hbm, ones_hbm, zf_hbm,
                    dego_hbm, degi_hbm, sidx, didx, ones_v, acc, ssem):
        c = lax.axis_index("c")
        s = lax.axis_index("s")
        wid = s * NC + c
        pltpu.sync_copy(zf_hbm, acc.at[pl.ds(s * RPT, RPT)])
        pltpu.sync_copy(ones_hbm, ones_v)
        pltpu.sync_copy(src_hbm.at[wid], sidx)
        pltpu.sync_copy(dst_hbm.at[wid], didx)
        plsc.subcore_barrier()

        def scatter_phase(idx):
            for b in range(NBUF):
                pltpu.async_copy(ones_v, acc.at[idx.at[b]], ssem.at[b],
                                 add=True)

            def grp(g, carry):
                for b in range(NBUF):
                    j = (g + 1) * NBUF + b
                    pltpu.make_async_copy(
                        ones_v, acc.at[idx.at[j]], ssem.at[b]).wait()
                    pltpu.async_copy(
                        ones_v, acc.at[idx.at[j]], ssem.at[b], add=True)
                return carry

            lax.fori_loop(0, NGROUP - 1, grp, 0)
            for b in range(NBUF):
                pltpu.make_async_copy(
                    ones_v, acc.at[idx.at[b]], ssem.at[b]).wait()

        scatter_phase(sidx)
        plsc.subcore_barrier()
        pltpu.sync_copy(acc.at[pl.ds(s * RPT, RPT)],
                        dego_hbm.at[c, pl.ds(s * RPT, RPT)])
        pltpu.sync_copy(zf_hbm, acc.at[pl.ds(s * RPT, RPT)])
        plsc.subcore_barrier()
        scatter_phase(didx)
        plsc.subcore_barrier()
        pltpu.sync_copy(acc.at[pl.ds(s * RPT, RPT)],
                        degi_hbm.at[c, pl.ds(s * RPT, RPT)])

    return _deg_kernel


@functools.cache
def _get_agg_kernel():
    @functools.partial(
        pl.kernel,
        out_type=jax.ShapeDtypeStruct((NC, NP, H), jnp.float32),
        mesh=_mesh(),
        scratch_types=[
            pltpu.VMEM((NGCHUNK, CHUNK), jnp.int32),
            pltpu.VMEM((NCHUNK, CHUNK), jnp.int32),
            pltpu.VMEM((NBUF, CHUNK, H), jnp.float32),
            pltpu.VMEM_SHARED((NPS, H), jnp.float32),
            pltpu.SemaphoreType.DMA((NBUF,)),
            pltpu.SemaphoreType.DMA((NBUF,)),
        ],
    )
    def _agg_kernel(hw_hbm, src_hbm, dst_hbm, zf_hbm,
                    out_hbm, sidx, didx, rows, acc, gsem, ssem):
        c = lax.axis_index("c")
        s = lax.axis_index("s")
        wid = s * NC + c
        pltpu.sync_copy(zf_hbm, acc.at[pl.ds(s * RPT, RPT)])
        pltpu.sync_copy(src_hbm.at[wid], sidx)
        pltpu.sync_copy(dst_hbm.at[wid], didx)
        plsc.subcore_barrier()

        for b in range(NBUF):
            pltpu.async_copy(hw_hbm.at[sidx.at[b]], rows.at[b], gsem.at[b])

        def grp(g, carry):
            for b in range(NBUF):
                j = g * NBUF + b
                pltpu.make_async_copy(
                    hw_hbm.at[sidx.at[j]], rows.at[b], gsem.at[b]).wait()
                pltpu.async_copy(
                    rows.at[b], acc.at[didx.at[j]], ssem.at[b], add=True)
                pltpu.make_async_copy(
                    rows.at[b], acc.at[didx.at[j]], ssem.at[b]).wait()
                pltpu.async_copy(
                    hw_hbm.at[sidx.at[j + NBUF]], rows.at[b], gsem.at[b])
            return carry

        lax.fori_loop(0, NGROUP, grp, 0)
        for b in range(NBUF):
            pltpu.make_async_copy(
                hw_hbm.at[sidx.at[NCHUNK + b]], rows.at[b], gsem.at[b]).wait()
        plsc.subcore_barrier()
        pltpu.sync_copy(acc.at[pl.ds(s * RPT, RPT)],
                        out_hbm.at[c, pl.ds(s * RPT, RPT)])

    return _agg_kernel


def _norm_col(degp):
    return lax.rsqrt(jnp.maximum((degp[0] + degp[1])[:, :1], 1.0))


def _tc_pre_body(x_ref, w_ref, dego_ref, hw_ref):
    ns = _norm_col(dego_ref[...])
    hw_ref[...] = (
        jnp.dot(x_ref[...], w_ref[...], preferred_element_type=jnp.float32) * ns
    )


def _tc_mid_body(accp_ref, degi_ref, b_ref, w_ref, dego_ref, h_ref, hw_ref):
    acc = accp_ref[0] + accp_ref[1]
    nd = _norm_col(degi_ref[...])
    h = jnp.maximum(acc * nd + b_ref[...], 0.0)
    h_ref[...] = h
    ns = _norm_col(dego_ref[...])
    hw_ref[...] = (
        jnp.dot(h, w_ref[...], preferred_element_type=jnp.float32) * ns
    )


def _tc_fin_body(accp_ref, degi_ref, b_ref, h0_ref, h1_ref, h2_ref,
                 wl_ref, bl_ref, out_ref):
    acc = accp_ref[0] + accp_ref[1]
    nd = _norm_col(degi_ref[...])
    h3 = jnp.maximum(acc * nd + b_ref[...], 0.0)
    wl = wl_ref[...]
    out = jnp.dot(h0_ref[...], wl[0:H], preferred_element_type=jnp.float32)
    out += jnp.dot(h1_ref[...], wl[H:2 * H], preferred_element_type=jnp.float32)
    out += jnp.dot(h2_ref[...], wl[2 * H:3 * H], preferred_element_type=jnp.float32)
    out += jnp.dot(h3, wl[3 * H:4 * H], preferred_element_type=jnp.float32)
    out_ref[...] = out + bl_ref[...]


_GRID = NP // BN

_feat_spec = pl.BlockSpec((BN, H), lambda i: (i, 0))
_pair_spec = pl.BlockSpec((2, BN, H), lambda i: (0, i, 0))
_deg_spec = pl.BlockSpec((2, BN, H), lambda i: (0, i, 0))
_w_spec = pl.BlockSpec((H, H), lambda i: (0, 0))
_b_spec = pl.BlockSpec((1, H), lambda i: (0, 0))

_tc_pre = pl.pallas_call(
    _tc_pre_body,
    grid=(_GRID,),
    in_specs=[_feat_spec, _w_spec, _deg_spec],
    out_specs=_feat_spec,
    out_shape=jax.ShapeDtypeStruct((NP, H), jnp.float32),
)

_tc_mid = pl.pallas_call(
    _tc_mid_body,
    grid=(_GRID,),
    in_specs=[_pair_spec, _deg_spec, _b_spec, _w_spec, _deg_spec],
    out_specs=[_feat_spec, _feat_spec],
    out_shape=[
        jax.ShapeDtypeStruct((NP, H), jnp.float32),
        jax.ShapeDtypeStruct((NP, H), jnp.float32),
    ],
)

_tc_fin = pl.pallas_call(
    _tc_fin_body,
    grid=(_GRID,),
    in_specs=[_pair_spec, _deg_spec, _b_spec, _feat_spec, _feat_spec,
              _feat_spec, pl.BlockSpec((4 * H, H), lambda i: (0, 0)), _b_spec],
    out_specs=_feat_spec,
    out_shape=jax.ShapeDtypeStruct((NP, H), jnp.float32),
)


def kernel(x, edge_index, W0, b0, W1, b1, W2, b2, W3, b3, Wl, bl):
    f32 = jnp.float32
    xp = jnp.zeros((NP, D), f32).at[:N].set(x)
    pad = N + jnp.arange(EPAD - E, dtype=jnp.int32) % (NPS - N)
    look = (N + jnp.arange(NW * NBUF * CHUNK, dtype=jnp.int32) % (NPS - N)
            ).reshape(NW, NBUF, CHUNK)
    srcp = jnp.concatenate([
        jnp.concatenate([edge_index[0], pad]).reshape(NW, NCHUNK, CHUNK),
        look], axis=1)
    dstp = jnp.concatenate([edge_index[1], pad]).reshape(NW, NCHUNK, CHUNK)
    zf = jnp.zeros((RPT, H), f32)
    ones128 = jnp.ones((CHUNK, H), f32)
    wlp = jnp.zeros((4 * H, H), f32).at[:, :C].set(Wl)
    blp = jnp.zeros((1, H), f32).at[0, :C].set(bl)

    _deg_kernel = _get_deg_kernel()
    _agg_kernel = _get_agg_kernel()
    dego, degi = _deg_kernel(srcp, dstp, ones128, zf)

    hs = []
    hw = _tc_pre(xp, W0, dego)
    for (bprev, W) in ((b0, W1), (b1, W2), (b2, W3)):
        accp = _agg_kernel(hw, srcp, dstp, zf)
        h, hw = _tc_mid(accp, degi, bprev.reshape(1, H), W, dego)
        hs.append(h)
    accp = _agg_kernel(hw, srcp, dstp, zf)
    outp = _tc_fin(accp, degi, b3.reshape(1, H), hs[0], hs[1], hs[2], wlp, blp)
    return outp[:N, :C]
